# Initial kernel scaffold; baseline (speedup 1.0000x reference)
import jax
import jax.numpy as jnp
from jax.experimental import pallas as pl


def kernel(hidden_states, cls_attn, dominant_num):
    raise NotImplementedError("write your pallas kernel here")


# speedup vs baseline: 1.4583x; 1.4583x over previous
import jax
import jax.numpy as jnp
from jax.experimental import pallas as pl
from jax.experimental.pallas import tpu as pltpu

SEL = 64


def _scope_kernel(nsel_ref, hid_ref, cls_ref, tok_ref, idx_ref,
                  cos_ref, sel_ref, cmax_ref, idxr_ref, idxc_ref):
    hid = hid_ref[0]
    n_tok = hid.shape[0] - 1
    feat = hid[1:, :]

    nrm = jnp.sqrt(jnp.sum(feat * feat, axis=1, keepdims=True))
    normf = feat / nrm
    cos_ref[...] = jax.lax.dot_general(
        normf, normf, (((1,), (1,)), ((), ())),
        preferred_element_type=jnp.float32)

    clsp = cls_ref[0]
    nsel = nsel_ref[0, 0]
    row_iota = jax.lax.broadcasted_iota(jnp.int32, (n_tok, 1), 0)
    lane_k = jax.lax.broadcasted_iota(jnp.int32, (1, SEL), 1)
    col_k = jax.lax.broadcasted_iota(jnp.int32, (SEL, 1), 0)

    sel_ref[...] = jnp.zeros((n_tok, 1), dtype=jnp.float32)
    cmax_ref[...] = jnp.zeros((1, n_tok), dtype=jnp.float32)
    idxr_ref[...] = jnp.zeros((1, SEL), dtype=jnp.int32)
    idxc_ref[...] = jnp.zeros((SEL, 1), dtype=jnp.int32)

    def body(i, _):
        selected = sel_ref[...]
        cur_max = cmax_ref[...]
        g = jnp.sum(jnp.maximum(cos_ref[...] - cur_max, 0.0),
                    axis=1, keepdims=True)
        g = g * clsp
        g = jnp.where(selected > 0.0, -jnp.inf, g)
        m = jnp.max(g)
        best = jnp.min(jnp.where(g == m, row_iota, n_tok))
        active = i < nsel
        sel_ref[...] = jnp.where(
            active & (row_iota == best), 1.0, selected)
        idxr_ref[...] = jnp.where(
            active & (lane_k == i), best, idxr_ref[...])
        idxc_ref[...] = jnp.where(
            active & (col_k == i), best, idxc_ref[...])
        new_max = jnp.maximum(cur_max, cos_ref[pl.ds(best, 1), :])
        cmax_ref[...] = jnp.where(active, new_max, cur_max)
        return 0

    jax.lax.fori_loop(0, SEL, body, 0)
    idx_row = idxr_ref[...]
    idx_col = idxc_ref[...]

    idx_ref[0] = idx_row + 1

    cmp = (idx_col < idx_row) | ((idx_col == idx_row) & (col_k < lane_k))
    rank_row = jnp.sum(cmp.astype(jnp.int32), axis=0, keepdims=True)
    perm = (rank_row == col_k)
    sorted_col = jnp.sum(jnp.where(perm, idx_row, 0),
                         axis=1, keepdims=True)

    lane_n = jax.lax.broadcasted_iota(jnp.int32, (1, n_tok), 1)
    onehot = (sorted_col == lane_n).astype(jnp.float32)
    tok_ref[0] = jax.lax.dot_general(
        onehot, feat, (((1,), (0,)), ((), ())),
        preferred_element_type=jnp.float32,
        precision=jax.lax.Precision.HIGHEST)


def kernel(hidden_states, cls_attn, dominant_num):
    B, N1, D = hidden_states.shape
    N = N1 - 1
    nsel = jnp.asarray(dominant_num, jnp.int32).reshape(1, 1)
    cls_col = cls_attn[:, :, None]
    tok, idx = pl.pallas_call(
        _scope_kernel,
        grid=(B,),
        in_specs=[
            pl.BlockSpec(memory_space=pltpu.SMEM),
            pl.BlockSpec((1, N1, D), lambda b: (b, 0, 0)),
            pl.BlockSpec((1, N, 1), lambda b: (b, 0, 0)),
        ],
        out_specs=[
            pl.BlockSpec((1, SEL, D), lambda b: (b, 0, 0)),
            pl.BlockSpec((1, 1, SEL), lambda b: (b, 0, 0)),
        ],
        out_shape=[
            jax.ShapeDtypeStruct((B, SEL, D), jnp.float32),
            jax.ShapeDtypeStruct((B, 1, SEL), jnp.int32),
        ],
        scratch_shapes=[
            pltpu.VMEM((N, N), jnp.float32),
            pltpu.VMEM((N, 1), jnp.float32),
            pltpu.VMEM((1, N), jnp.float32),
            pltpu.VMEM((1, SEL), jnp.int32),
            pltpu.VMEM((SEL, 1), jnp.int32),
        ],
        compiler_params=pltpu.CompilerParams(
            dimension_semantics=("parallel",)),
    )(nsel, hidden_states, cls_col)
    return tok, idx.reshape(B, SEL)
